# baseline (speedup 1.0000x reference)
import jax
import jax.numpy as jnp
from jax import lax
from jax.experimental import pallas as pl
from jax.experimental.pallas import tpu as pltpu

F32 = jnp.float32
BF16 = jnp.bfloat16


def _mm(a, b):
    return lax.dot_general(a, b, (((1,), (0,)), ((), ())),
                           preferred_element_type=F32)


def _mm_t(a, b):
    return lax.dot_general(a, b, (((0,), (0,)), ((), ())),
                           preferred_element_type=F32)


def _sub_body(xp_ref, et_ref, ws0_ref, wd0_ref, wr0_ref, am0_ref,
              ws1_ref, wd1_ref, am1_ref, wg_ref, wl_ref, em_ref, bp_ref,
              out_ref):
    bp = bp_ref[...]
    b_src1 = bp[0:1, :]
    b_dst1 = bp[1:2, :]
    b_gate = bp[2:3, 0:1]
    b_lin = bp[3:4, 0:out_ref.shape[2]]
    for k in range(xp_ref.shape[0]):
        _one_subgraph(k, xp_ref, et_ref, ws0_ref, wd0_ref, wr0_ref, am0_ref,
                      ws1_ref, wd1_ref, am1_ref, wg_ref, wl_ref, em_ref,
                      b_src1, b_dst1, b_gate, b_lin, out_ref)


def _one_subgraph(k, xp_ref, et_ref, ws0_ref, wd0_ref, wr0_ref, am0_ref,
                  ws1_ref, wd1_ref, am1_ref, wg_ref, wl_ref, em_ref,
                  b_src1, b_dst1, b_gate, b_lin, out_ref):
    xp = xp_ref[k]
    et = et_ref[k]
    n_nodes = xp.shape[0]
    n_edges = et.shape[0]
    src = et[:, 0:1]
    dst = et[:, 1:2]
    n_iota = lax.broadcasted_iota(jnp.int32, (n_edges, n_nodes), 1)
    oh_src = (src == n_iota).astype(BF16)
    oh_dst = (dst == n_iota).astype(BF16)

    def gat_layer(fs, fd, res, am_ref):
        fs_src = _mm(oh_src, fs)
        fd_dst = _mm(oh_dst, fd)
        es = fs_src + fd_dst
        e = jnp.where(es >= 0, es, es * 0.2)
        logits_t = lax.dot_general(am_ref[...], e.astype(BF16),
                                   (((0,), (1,)), ((), ())),
                                   preferred_element_type=F32)
        gmax = jnp.max(logits_t, axis=(0, 1), keepdims=True)
        exl_t = jnp.exp(logits_t - gmax).astype(BF16)
        denom_t = _mm(exl_t, oh_dst)
        denom_dst_t = lax.dot_general(
            denom_t.astype(BF16), oh_dst, (((1,), (1,)), ((), ())),
            preferred_element_type=F32)
        a_t = exl_t / jnp.maximum(denom_dst_t, 1e-9)
        a_exp = _mm_t(a_t.astype(BF16), em_ref[...])
        rst = _mm_t(oh_dst, (a_exp * fs_src).astype(BF16))
        return jnp.maximum(rst + res, 0.0)

    def pool(h):
        cmax = jnp.max(h, axis=0, keepdims=True)
        ex = jnp.exp(h - cmax)
        newh = (ex * (1.0 / jnp.sum(ex, axis=0, keepdims=True))).astype(BF16)
        g_t = lax.dot_general(wg_ref[...], newh,
                              (((0,), (1,)), ((), ())),
                              preferred_element_type=F32) + b_gate
        gmax = jnp.max(g_t, axis=1, keepdims=True)
        gex = jnp.exp(g_t - gmax)
        gate_t = (gex * (1.0 / jnp.sum(gex, axis=1, keepdims=True))
                  ).astype(BF16)
        return _mm(gate_t, newh)

    fs0 = _mm(xp, ws0_ref[...])
    fd0 = _mm(xp, wd0_ref[...])
    res0 = _mm(xp, wr0_ref[...])
    h1 = gat_layer(fs0.astype(BF16), fd0.astype(BF16), res0, am0_ref)
    hg = pool(h1)
    h1b = h1.astype(BF16)
    fs1 = (_mm(h1b, ws1_ref[...]) + b_src1).astype(BF16)
    fd1 = (_mm(h1b, wd1_ref[...]) + b_dst1).astype(BF16)
    h2 = gat_layer(fs1, fd1, h1, am1_ref)
    hg = hg + pool(h2)
    out_ref[k] = _mm(hg.astype(BF16), wl_ref[...]) + b_lin


def _global_body(ge_ref, gf_ref, tf_ref, gnf_ref, wgcn_ref, wl2a_ref,
                 wl2b_ref, wclsa_ref, wclsb_ref, bp_ref, out_ref):
    ge = ge_ref[...]
    gf = gf_ref[...]
    gn = gf.shape[0]
    n_edges = ge.shape[0]
    chunk = 512
    n_chunks = n_edges // chunk
    bp = bp_ref[...]
    b_gcn = bp[0:1, 0:wgcn_ref.shape[1]]
    b_l2 = bp[1:2, 0:wl2a_ref.shape[1]]
    b_cls = bp[2:3, 0:out_ref.shape[1]]

    def onehots(c):
        sc = ge[c * chunk:(c + 1) * chunk, 0:1]
        dc = ge[c * chunk:(c + 1) * chunk, 1:2]
        n_iota = lax.broadcasted_iota(jnp.int32, (chunk, gn), 1)
        return (sc == n_iota).astype(F32), (dc == n_iota).astype(F32)

    ones_col = jnp.ones((chunk, 1), F32)
    deg_o = jnp.zeros((gn, 1), F32)
    deg_i = jnp.zeros((gn, 1), F32)
    for c in range(n_chunks):
        ohs, ohd = onehots(c)
        deg_o = deg_o + _mm_t(ohs, ones_col)
        deg_i = deg_i + _mm_t(ohd, ones_col)
    rsq_o = lax.rsqrt(jnp.maximum(deg_o, 1.0))
    rsq_i = lax.rsqrt(jnp.maximum(deg_i, 1.0))
    hsrc = gf * rsq_o
    agg = jnp.zeros_like(gf)
    for c in range(n_chunks):
        ohs, ohd = onehots(c)
        agg = agg + _mm_t(ohd, _mm(ohs, hsrc))
    agg = agg * rsq_i
    gcn = jnp.maximum(_mm(agg, wgcn_ref[...]) + b_gcn, 0.0)
    tra = _mm(gnf_ref[...], wl2a_ref[...]) + _mm(tf_ref[...], wl2b_ref[...]) + b_l2
    out_ref[...] = _mm(tra, wclsa_ref[...]) + _mm(gcn, wclsb_ref[...]) + b_cls


def _attn_mask(attn):
    n_heads, head_dim = attn.shape
    hd = n_heads * head_dim
    idx = jnp.arange(hd)
    return jnp.zeros((hd, n_heads), F32).at[idx, idx // head_dim].set(
        attn.reshape(-1))


def _full_spec(shape):
    nd = len(shape)
    return pl.BlockSpec(shape, lambda *_, _nd=nd: (0,) * _nd)


def kernel(sub_x, sub_edge_index, g_edge_index, g_feat, traFeat, params):
    p = params
    s, n, f_in = sub_x.shape
    e = sub_edge_index.shape[2]
    hd = p['W_src0'].shape[1]
    out_dim = p['W_lin'].shape[1]

    n_heads = p['attn0'].shape[0]
    xp = jnp.concatenate([sub_x, jnp.ones((s, n, 1), F32)],
                         axis=-1).astype(BF16)
    et = jnp.transpose(sub_edge_index.astype(jnp.int32), (0, 2, 1))
    ws0 = jnp.concatenate([p['W_src0'], p['b_src0'][None, :]],
                          axis=0).astype(BF16)
    wd0 = jnp.concatenate([p['W_dst0'], p['b_dst0'][None, :]],
                          axis=0).astype(BF16)
    wr0 = jnp.concatenate([p['res_W0'], p['res_b0'][None, :]],
                          axis=0).astype(BF16)
    am0 = _attn_mask(p['attn0']).astype(BF16)
    am1 = _attn_mask(p['attn1']).astype(BF16)
    em = (jnp.arange(hd)[None, :] // (hd // n_heads)
          == jnp.arange(n_heads)[:, None]).astype(BF16)
    bp = jnp.zeros((8, hd), F32)
    bp = bp.at[0, :].set(p['b_src1'])
    bp = bp.at[1, :].set(p['b_dst1'])
    bp = bp.at[2, 0].set(p['b_gate'][0])
    bp = bp.at[3, 0:out_dim].set(p['b_lin'])

    blk = 2
    gnf = pl.pallas_call(
        _sub_body,
        grid=(s // blk,),
        in_specs=[
            pl.BlockSpec((blk, n, f_in + 1), lambda i: (i, 0, 0)),
            pl.BlockSpec((blk, e, 2), lambda i: (i, 0, 0)),
            _full_spec(ws0.shape), _full_spec(wd0.shape),
            _full_spec(wr0.shape), _full_spec(am0.shape),
            _full_spec(p['W_src1'].shape), _full_spec(p['W_dst1'].shape),
            _full_spec(am1.shape), _full_spec(p['W_gate'].shape),
            _full_spec(p['W_lin'].shape), _full_spec(em.shape),
            _full_spec(bp.shape),
        ],
        out_specs=pl.BlockSpec((blk, 1, out_dim), lambda i: (i, 0, 0)),
        out_shape=jax.ShapeDtypeStruct((s, 1, out_dim), F32),
    )(xp, et, ws0, wd0, wr0, am0, p['W_src1'].astype(BF16),
      p['W_dst1'].astype(BF16), am1, p['W_gate'].astype(BF16),
      p['W_lin'].astype(BF16), em, bp)
    gnf = gnf.reshape(s, out_dim)

    gn, td = g_feat.shape
    geT = jnp.transpose(g_edge_index.astype(jnp.int32), (1, 0))
    wl2a = p['W_l2'][:out_dim, :]
    wl2b = p['W_l2'][out_dim:, :]
    h_dim = wl2a.shape[1]
    wclsa = p['W_cls'][:h_dim, :]
    wclsb = p['W_cls'][h_dim:, :]
    bp2 = jnp.zeros((4, max(td, h_dim)), F32)
    bp2 = bp2.at[0, 0:td].set(p['b_gcn'])
    bp2 = bp2.at[1, 0:h_dim].set(p['b_l2'])
    bp2 = bp2.at[2, 0:2].set(p['b_cls'])

    out = pl.pallas_call(
        _global_body,
        in_specs=[_full_spec(geT.shape), _full_spec(g_feat.shape),
                  _full_spec(traFeat.shape), _full_spec(gnf.shape),
                  _full_spec(p['W_gcn'].shape), _full_spec(wl2a.shape),
                  _full_spec(wl2b.shape), _full_spec(wclsa.shape),
                  _full_spec(wclsb.shape), _full_spec(bp2.shape)],
        out_specs=_full_spec((gn, 2)),
        out_shape=jax.ShapeDtypeStruct((gn, 2), F32),
    )(geT, g_feat, traFeat, gnf, p['W_gcn'], wl2a, wl2b, wclsa, wclsb, bp2)
    return out


# speedup vs baseline: 1.0079x; 1.0079x over previous
import jax
import jax.numpy as jnp
from jax import lax
from jax.experimental import pallas as pl
from jax.experimental.pallas import tpu as pltpu

F32 = jnp.float32
BF16 = jnp.bfloat16


def _mm(a, b):
    return lax.dot_general(a, b, (((1,), (0,)), ((), ())),
                           preferred_element_type=F32)


def _mm_t(a, b):
    return lax.dot_general(a, b, (((0,), (0,)), ((), ())),
                           preferred_element_type=F32)


def _sub_body(xp_ref, et_ref, ws0_ref, wd0_ref, wr0_ref, am0_ref,
              ws1_ref, wd1_ref, am1_ref, wg_ref, wl_ref, em_ref, bp_ref,
              out_ref):
    bp = bp_ref[...]
    b_src1 = bp[0:1, :]
    b_dst1 = bp[1:2, :]
    b_gate = bp[2:3, 0:1]
    b_lin = bp[3:4, 0:out_ref.shape[2]]
    for k in range(xp_ref.shape[0]):
        _one_subgraph(k, xp_ref, et_ref, ws0_ref, wd0_ref, wr0_ref, am0_ref,
                      ws1_ref, wd1_ref, am1_ref, wg_ref, wl_ref, em_ref,
                      b_src1, b_dst1, b_gate, b_lin, out_ref)


def _one_subgraph(k, xp_ref, et_ref, ws0_ref, wd0_ref, wr0_ref, am0_ref,
                  ws1_ref, wd1_ref, am1_ref, wg_ref, wl_ref, em_ref,
                  b_src1, b_dst1, b_gate, b_lin, out_ref):
    xp = xp_ref[k]
    et = et_ref[k]
    n_nodes = xp.shape[0]
    n_edges = et.shape[0]
    src = et[:, 0:1]
    dst = et[:, 1:2]
    n_iota = lax.broadcasted_iota(jnp.int32, (n_edges, n_nodes), 1)
    oh_src = (src == n_iota).astype(BF16)
    oh_dst = (dst == n_iota).astype(BF16)

    def gat_layer(fs, fd, res, am_ref):
        fs_src = _mm(oh_src, fs)
        fd_dst = _mm(oh_dst, fd)
        es = fs_src + fd_dst
        e = jnp.where(es >= 0, es, es * 0.2)
        logits_t = lax.dot_general(am_ref[...], e.astype(BF16),
                                   (((0,), (1,)), ((), ())),
                                   preferred_element_type=F32)
        gmax = jnp.max(logits_t, axis=(0, 1), keepdims=True)
        exl_t = jnp.exp(logits_t - gmax).astype(BF16)
        denom_t = _mm(exl_t, oh_dst)
        denom_dst_t = lax.dot_general(
            denom_t.astype(BF16), oh_dst, (((1,), (1,)), ((), ())),
            preferred_element_type=F32)
        a_t = exl_t / jnp.maximum(denom_dst_t, 1e-9)
        a_exp = _mm_t(a_t.astype(BF16), em_ref[...])
        rst = _mm_t(oh_dst, (a_exp * fs_src).astype(BF16))
        return jnp.maximum(rst + res, 0.0)

    def pool(h):
        cmax = jnp.max(h, axis=0, keepdims=True)
        ex = jnp.exp(h - cmax)
        newh = (ex * (1.0 / jnp.sum(ex, axis=0, keepdims=True))).astype(BF16)
        g_t = lax.dot_general(wg_ref[...], newh,
                              (((0,), (1,)), ((), ())),
                              preferred_element_type=F32) + b_gate
        gmax = jnp.max(g_t, axis=1, keepdims=True)
        gex = jnp.exp(g_t - gmax)
        gate_t = (gex * (1.0 / jnp.sum(gex, axis=1, keepdims=True))
                  ).astype(BF16)
        return _mm(gate_t, newh)

    fs0 = _mm(xp, ws0_ref[...])
    fd0 = _mm(xp, wd0_ref[...])
    res0 = _mm(xp, wr0_ref[...])
    h1 = gat_layer(fs0.astype(BF16), fd0.astype(BF16), res0, am0_ref)
    hg = pool(h1)
    h1b = h1.astype(BF16)
    fs1 = (_mm(h1b, ws1_ref[...]) + b_src1).astype(BF16)
    fd1 = (_mm(h1b, wd1_ref[...]) + b_dst1).astype(BF16)
    h2 = gat_layer(fs1, fd1, h1, am1_ref)
    hg = hg + pool(h2)
    out_ref[k] = _mm(hg.astype(BF16), wl_ref[...]) + b_lin


def _global_body(ge_ref, gf_ref, tf_ref, gnf_ref, wgcn_ref, wl2a_ref,
                 wl2b_ref, wclsa_ref, wclsb_ref, bp_ref, out_ref):
    ge = ge_ref[...]
    gf = gf_ref[...]
    gn = gf.shape[0]
    n_edges = ge.shape[0]
    chunk = 512
    n_chunks = n_edges // chunk
    bp = bp_ref[...]
    b_gcn = bp[0:1, 0:wgcn_ref.shape[1]]
    b_l2 = bp[1:2, 0:wl2a_ref.shape[1]]
    b_cls = bp[2:3, 0:out_ref.shape[1]]

    def onehots(c):
        sc = ge[c * chunk:(c + 1) * chunk, 0:1]
        dc = ge[c * chunk:(c + 1) * chunk, 1:2]
        n_iota = lax.broadcasted_iota(jnp.int32, (chunk, gn), 1)
        return (sc == n_iota).astype(F32), (dc == n_iota).astype(F32)

    ones_col = jnp.ones((chunk, 1), F32)
    deg_o = jnp.zeros((gn, 1), F32)
    deg_i = jnp.zeros((gn, 1), F32)
    for c in range(n_chunks):
        ohs, ohd = onehots(c)
        deg_o = deg_o + _mm_t(ohs, ones_col)
        deg_i = deg_i + _mm_t(ohd, ones_col)
    rsq_o = lax.rsqrt(jnp.maximum(deg_o, 1.0))
    rsq_i = lax.rsqrt(jnp.maximum(deg_i, 1.0))
    hsrc = gf * rsq_o
    agg = jnp.zeros_like(gf)
    for c in range(n_chunks):
        ohs, ohd = onehots(c)
        agg = agg + _mm_t(ohd, _mm(ohs, hsrc))
    agg = agg * rsq_i
    gcn = jnp.maximum(_mm(agg, wgcn_ref[...]) + b_gcn, 0.0)
    tra = _mm(gnf_ref[...], wl2a_ref[...]) + _mm(tf_ref[...], wl2b_ref[...]) + b_l2
    out_ref[...] = _mm(tra, wclsa_ref[...]) + _mm(gcn, wclsb_ref[...]) + b_cls


def _attn_mask(attn):
    n_heads, head_dim = attn.shape
    hd = n_heads * head_dim
    idx = jnp.arange(hd)
    return jnp.zeros((hd, n_heads), F32).at[idx, idx // head_dim].set(
        attn.reshape(-1))


def _full_spec(shape):
    nd = len(shape)
    return pl.BlockSpec(shape, lambda *_, _nd=nd: (0,) * _nd)


def kernel(sub_x, sub_edge_index, g_edge_index, g_feat, traFeat, params):
    p = params
    s, n, f_in = sub_x.shape
    e = sub_edge_index.shape[2]
    hd = p['W_src0'].shape[1]
    out_dim = p['W_lin'].shape[1]

    n_heads = p['attn0'].shape[0]
    xp = jnp.concatenate([sub_x, jnp.ones((s, n, 1), F32)],
                         axis=-1).astype(BF16)
    et = jnp.transpose(sub_edge_index.astype(jnp.int32), (0, 2, 1))
    ws0 = jnp.concatenate([p['W_src0'], p['b_src0'][None, :]],
                          axis=0).astype(BF16)
    wd0 = jnp.concatenate([p['W_dst0'], p['b_dst0'][None, :]],
                          axis=0).astype(BF16)
    wr0 = jnp.concatenate([p['res_W0'], p['res_b0'][None, :]],
                          axis=0).astype(BF16)
    am0 = _attn_mask(p['attn0']).astype(BF16)
    am1 = _attn_mask(p['attn1']).astype(BF16)
    em = (jnp.arange(hd)[None, :] // (hd // n_heads)
          == jnp.arange(n_heads)[:, None]).astype(BF16)
    bp = jnp.zeros((8, hd), F32)
    bp = bp.at[0, :].set(p['b_src1'])
    bp = bp.at[1, :].set(p['b_dst1'])
    bp = bp.at[2, 0].set(p['b_gate'][0])
    bp = bp.at[3, 0:out_dim].set(p['b_lin'])

    blk = 4
    gnf = pl.pallas_call(
        _sub_body,
        grid=(s // blk,),
        in_specs=[
            pl.BlockSpec((blk, n, f_in + 1), lambda i: (i, 0, 0)),
            pl.BlockSpec((blk, e, 2), lambda i: (i, 0, 0)),
            _full_spec(ws0.shape), _full_spec(wd0.shape),
            _full_spec(wr0.shape), _full_spec(am0.shape),
            _full_spec(p['W_src1'].shape), _full_spec(p['W_dst1'].shape),
            _full_spec(am1.shape), _full_spec(p['W_gate'].shape),
            _full_spec(p['W_lin'].shape), _full_spec(em.shape),
            _full_spec(bp.shape),
        ],
        out_specs=pl.BlockSpec((blk, 1, out_dim), lambda i: (i, 0, 0)),
        out_shape=jax.ShapeDtypeStruct((s, 1, out_dim), F32),
    )(xp, et, ws0, wd0, wr0, am0, p['W_src1'].astype(BF16),
      p['W_dst1'].astype(BF16), am1, p['W_gate'].astype(BF16),
      p['W_lin'].astype(BF16), em, bp)
    gnf = gnf.reshape(s, out_dim)

    gn, td = g_feat.shape
    geT = jnp.transpose(g_edge_index.astype(jnp.int32), (1, 0))
    wl2a = p['W_l2'][:out_dim, :]
    wl2b = p['W_l2'][out_dim:, :]
    h_dim = wl2a.shape[1]
    wclsa = p['W_cls'][:h_dim, :]
    wclsb = p['W_cls'][h_dim:, :]
    bp2 = jnp.zeros((4, max(td, h_dim)), F32)
    bp2 = bp2.at[0, 0:td].set(p['b_gcn'])
    bp2 = bp2.at[1, 0:h_dim].set(p['b_l2'])
    bp2 = bp2.at[2, 0:2].set(p['b_cls'])

    out = pl.pallas_call(
        _global_body,
        in_specs=[_full_spec(geT.shape), _full_spec(g_feat.shape),
                  _full_spec(traFeat.shape), _full_spec(gnf.shape),
                  _full_spec(p['W_gcn'].shape), _full_spec(wl2a.shape),
                  _full_spec(wl2b.shape), _full_spec(wclsa.shape),
                  _full_spec(wclsb.shape), _full_spec(bp2.shape)],
        out_specs=_full_spec((gn, 2)),
        out_shape=jax.ShapeDtypeStruct((gn, 2), F32),
    )(geT, g_feat, traFeat, gnf, p['W_gcn'], wl2a, wl2b, wclsa, wclsb, bp2)
    return out


# speedup vs baseline: 2.3082x; 2.2900x over previous
import jax
import jax.numpy as jnp
from jax import lax
from jax.experimental import pallas as pl
from jax.experimental.pallas import tpu as pltpu

F32 = jnp.float32
BF16 = jnp.bfloat16


def _mm(a, b):
    return lax.dot_general(a, b, (((1,), (0,)), ((), ())),
                           preferred_element_type=F32)


def _mm_t(a, b):
    return lax.dot_general(a, b, (((0,), (0,)), ((), ())),
                           preferred_element_type=F32)


def _smap(f, *ls):
    return [f(*xs) for xs in zip(*ls)]


def _sub_body(xp_ref, et_ref, ws0_ref, wd0_ref, wr0_ref, am0_ref,
              ws1_ref, wd1_ref, am1_ref, wg_ref, wl_ref, em_ref, bp_ref,
              out_ref):
    bp = bp_ref[...]
    b_src1 = bp[0:1, :]
    b_dst1 = bp[1:2, :]
    b_gate = bp[2:3, 0:1]
    b_lin = bp[3:4, 0:out_ref.shape[2]]
    blk = xp_ref.shape[0]
    n_nodes = xp_ref.shape[1]
    n_edges = et_ref.shape[1]
    ks = list(range(blk))

    xs = [xp_ref[k] for k in ks]
    n_iota = lax.broadcasted_iota(jnp.int32, (n_edges, n_nodes), 1)
    oh_src = [(et_ref[k][:, 0:1] == n_iota).astype(BF16) for k in ks]
    oh_dst = [(et_ref[k][:, 1:2] == n_iota).astype(BF16) for k in ks]

    def gat_layer(fss, fds, ress, am_ref):
        am = am_ref[...]
        fs_src = _smap(lambda o, f: _mm(o, f), oh_src, fss)
        fd_dst = _smap(lambda o, f: _mm(o, f), oh_dst, fds)
        e = _smap(lambda a, b: jnp.where(a + b >= 0, a + b, (a + b) * 0.2),
                  fs_src, fd_dst)
        eb = _smap(lambda x: x.astype(BF16), e)
        logits_t = _smap(
            lambda x: lax.dot_general(am, x, (((0,), (1,)), ((), ())),
                                      preferred_element_type=F32), eb)
        gmax = _smap(lambda l: jnp.max(l, axis=(0, 1), keepdims=True),
                     logits_t)
        exl_t = _smap(lambda l, m: jnp.exp(l - m).astype(BF16),
                      logits_t, gmax)
        denom_t = _smap(lambda x, o: _mm(x, o), exl_t, oh_dst)
        denom_dst_t = _smap(
            lambda d, o: lax.dot_general(d.astype(BF16), o,
                                         (((1,), (1,)), ((), ())),
                                         preferred_element_type=F32),
            denom_t, oh_dst)
        a_t = _smap(lambda x, d: (x / jnp.maximum(d, 1e-9)).astype(BF16),
                    exl_t, denom_dst_t)
        em = em_ref[...]
        a_exp = _smap(lambda a: _mm_t(a, em), a_t)
        wgt = _smap(lambda a, f: (a * f).astype(BF16), a_exp, fs_src)
        rst = _smap(lambda o, w: _mm_t(o, w), oh_dst, wgt)
        return _smap(lambda r, q: jnp.maximum(r + q, 0.0), rst, ress)

    def pool(hs):
        cmax = _smap(lambda h: jnp.max(h, axis=0, keepdims=True), hs)
        ex = _smap(lambda h, c: jnp.exp(h - c), hs, cmax)
        newh = _smap(
            lambda x: (x * (1.0 / jnp.sum(x, axis=0, keepdims=True))
                       ).astype(BF16), ex)
        wg = wg_ref[...]
        g_t = _smap(
            lambda nh: lax.dot_general(wg, nh, (((0,), (1,)), ((), ())),
                                       preferred_element_type=F32) + b_gate,
            newh)
        gmx = _smap(lambda g: jnp.max(g, axis=1, keepdims=True), g_t)
        gex = _smap(lambda g, m: jnp.exp(g - m), g_t, gmx)
        gate_t = _smap(
            lambda x: (x * (1.0 / jnp.sum(x, axis=1, keepdims=True))
                       ).astype(BF16), gex)
        return _smap(lambda g, nh: _mm(g, nh), gate_t, newh)

    ws0 = ws0_ref[...]
    wd0 = wd0_ref[...]
    wr0 = wr0_ref[...]
    fs0 = _smap(lambda x: _mm(x, ws0).astype(BF16), xs)
    fd0 = _smap(lambda x: _mm(x, wd0).astype(BF16), xs)
    res0 = _smap(lambda x: _mm(x, wr0), xs)
    h1 = gat_layer(fs0, fd0, res0, am0_ref)
    hg = pool(h1)
    h1b = _smap(lambda h: h.astype(BF16), h1)
    ws1 = ws1_ref[...]
    wd1 = wd1_ref[...]
    fs1 = _smap(lambda h: (_mm(h, ws1) + b_src1).astype(BF16), h1b)
    fd1 = _smap(lambda h: (_mm(h, wd1) + b_dst1).astype(BF16), h1b)
    h2 = gat_layer(fs1, fd1, h1, am1_ref)
    hg2 = pool(h2)
    wl = wl_ref[...]
    for k in ks:
        out_ref[k] = _mm((hg[k] + hg2[k]).astype(BF16), wl) + b_lin


def _global_body(ge_ref, gf_ref, tf_ref, gnf_ref, wgcn_ref, wl2a_ref,
                 wl2b_ref, wclsa_ref, wclsb_ref, bp_ref, out_ref):
    ge = ge_ref[...]
    gf = gf_ref[...]
    gn = gf.shape[0]
    n_edges = ge.shape[0]
    chunk = 512
    n_chunks = n_edges // chunk
    bp = bp_ref[...]
    b_gcn = bp[0:1, 0:wgcn_ref.shape[1]]
    b_l2 = bp[1:2, 0:wl2a_ref.shape[1]]
    b_cls = bp[2:3, 0:out_ref.shape[1]]

    def onehots(c):
        sc = ge[c * chunk:(c + 1) * chunk, 0:1]
        dc = ge[c * chunk:(c + 1) * chunk, 1:2]
        n_iota = lax.broadcasted_iota(jnp.int32, (chunk, gn), 1)
        return (sc == n_iota).astype(F32), (dc == n_iota).astype(F32)

    ones_col = jnp.ones((chunk, 1), F32)
    deg_o = jnp.zeros((gn, 1), F32)
    deg_i = jnp.zeros((gn, 1), F32)
    for c in range(n_chunks):
        ohs, ohd = onehots(c)
        deg_o = deg_o + _mm_t(ohs, ones_col)
        deg_i = deg_i + _mm_t(ohd, ones_col)
    rsq_o = lax.rsqrt(jnp.maximum(deg_o, 1.0))
    rsq_i = lax.rsqrt(jnp.maximum(deg_i, 1.0))
    hsrc = gf * rsq_o
    agg = jnp.zeros_like(gf)
    for c in range(n_chunks):
        ohs, ohd = onehots(c)
        agg = agg + _mm_t(ohd, _mm(ohs, hsrc))
    agg = agg * rsq_i
    gcn = jnp.maximum(_mm(agg, wgcn_ref[...]) + b_gcn, 0.0)
    tra = _mm(gnf_ref[...], wl2a_ref[...]) + _mm(tf_ref[...], wl2b_ref[...]) + b_l2
    out_ref[...] = _mm(tra, wclsa_ref[...]) + _mm(gcn, wclsb_ref[...]) + b_cls


def _attn_mask(attn):
    n_heads, head_dim = attn.shape
    hd = n_heads * head_dim
    idx = jnp.arange(hd)
    return jnp.zeros((hd, n_heads), F32).at[idx, idx // head_dim].set(
        attn.reshape(-1))


def _full_spec(shape):
    nd = len(shape)
    return pl.BlockSpec(shape, lambda *_, _nd=nd: (0,) * _nd)


def kernel(sub_x, sub_edge_index, g_edge_index, g_feat, traFeat, params):
    p = params
    s, n, f_in = sub_x.shape
    e = sub_edge_index.shape[2]
    hd = p['W_src0'].shape[1]
    out_dim = p['W_lin'].shape[1]

    n_heads = p['attn0'].shape[0]
    xp = jnp.concatenate([sub_x, jnp.ones((s, n, 1), F32)],
                         axis=-1).astype(BF16)
    et = jnp.transpose(sub_edge_index.astype(jnp.int32), (0, 2, 1))
    ws0 = jnp.concatenate([p['W_src0'], p['b_src0'][None, :]],
                          axis=0).astype(BF16)
    wd0 = jnp.concatenate([p['W_dst0'], p['b_dst0'][None, :]],
                          axis=0).astype(BF16)
    wr0 = jnp.concatenate([p['res_W0'], p['res_b0'][None, :]],
                          axis=0).astype(BF16)
    am0 = _attn_mask(p['attn0']).astype(BF16)
    am1 = _attn_mask(p['attn1']).astype(BF16)
    em = (jnp.arange(hd)[None, :] // (hd // n_heads)
          == jnp.arange(n_heads)[:, None]).astype(BF16)
    bp = jnp.zeros((8, hd), F32)
    bp = bp.at[0, :].set(p['b_src1'])
    bp = bp.at[1, :].set(p['b_dst1'])
    bp = bp.at[2, 0].set(p['b_gate'][0])
    bp = bp.at[3, 0:out_dim].set(p['b_lin'])

    blk = 4
    gnf = pl.pallas_call(
        _sub_body,
        grid=(s // blk,),
        in_specs=[
            pl.BlockSpec((blk, n, f_in + 1), lambda i: (i, 0, 0)),
            pl.BlockSpec((blk, e, 2), lambda i: (i, 0, 0)),
            _full_spec(ws0.shape), _full_spec(wd0.shape),
            _full_spec(wr0.shape), _full_spec(am0.shape),
            _full_spec(p['W_src1'].shape), _full_spec(p['W_dst1'].shape),
            _full_spec(am1.shape), _full_spec(p['W_gate'].shape),
            _full_spec(p['W_lin'].shape), _full_spec(em.shape),
            _full_spec(bp.shape),
        ],
        out_specs=pl.BlockSpec((blk, 1, out_dim), lambda i: (i, 0, 0)),
        out_shape=jax.ShapeDtypeStruct((s, 1, out_dim), F32),
    )(xp, et, ws0, wd0, wr0, am0, p['W_src1'].astype(BF16),
      p['W_dst1'].astype(BF16), am1, p['W_gate'].astype(BF16),
      p['W_lin'].astype(BF16), em, bp)
    gnf = gnf.reshape(s, out_dim)

    gn, td = g_feat.shape
    geT = jnp.transpose(g_edge_index.astype(jnp.int32), (1, 0))
    wl2a = p['W_l2'][:out_dim, :]
    wl2b = p['W_l2'][out_dim:, :]
    h_dim = wl2a.shape[1]
    wclsa = p['W_cls'][:h_dim, :]
    wclsb = p['W_cls'][h_dim:, :]
    bp2 = jnp.zeros((4, max(td, h_dim)), F32)
    bp2 = bp2.at[0, 0:td].set(p['b_gcn'])
    bp2 = bp2.at[1, 0:h_dim].set(p['b_l2'])
    bp2 = bp2.at[2, 0:2].set(p['b_cls'])

    out = pl.pallas_call(
        _global_body,
        in_specs=[_full_spec(geT.shape), _full_spec(g_feat.shape),
                  _full_spec(traFeat.shape), _full_spec(gnf.shape),
                  _full_spec(p['W_gcn'].shape), _full_spec(wl2a.shape),
                  _full_spec(wl2b.shape), _full_spec(wclsa.shape),
                  _full_spec(wclsb.shape), _full_spec(bp2.shape)],
        out_specs=_full_spec((gn, 2)),
        out_shape=jax.ShapeDtypeStruct((gn, 2), F32),
    )(geT, g_feat, traFeat, gnf, p['W_gcn'], wl2a, wl2b, wclsa, wclsb, bp2)
    return out


# speedup vs baseline: 2.5720x; 1.1143x over previous
import jax
import jax.numpy as jnp
from jax import lax
from jax.experimental import pallas as pl
from jax.experimental.pallas import tpu as pltpu

F32 = jnp.float32
BF16 = jnp.bfloat16


def _mm(a, b):
    return lax.dot_general(a, b, (((1,), (0,)), ((), ())),
                           preferred_element_type=F32)


def _mm_t(a, b):
    return lax.dot_general(a, b, (((0,), (0,)), ((), ())),
                           preferred_element_type=F32)


def _smap(f, *ls):
    return [f(*xs) for xs in zip(*ls)]


def _sub_body(xp_ref, et_ref, ws0_ref, wd0_ref, wr0_ref, am0_ref,
              ws1_ref, wd1_ref, am1_ref, wg_ref, wl_ref, em_ref, bp_ref,
              out_ref):
    bp = bp_ref[...]
    b_src1 = bp[0:1, :]
    b_dst1 = bp[1:2, :]
    b_gate = bp[2:3, 0:1]
    b_lin = bp[3:4, 0:out_ref.shape[2]]
    blk = xp_ref.shape[0]
    n_nodes = xp_ref.shape[1]
    n_edges = et_ref.shape[1]
    ks = list(range(blk))

    xs = [xp_ref[k] for k in ks]
    n_iota = lax.broadcasted_iota(jnp.int32, (n_edges, n_nodes), 1)
    oh_src = [(et_ref[k][:, 0:1] == n_iota).astype(BF16) for k in ks]
    oh_dst = [(et_ref[k][:, 1:2] == n_iota).astype(BF16) for k in ks]

    def gat_layer(fss, fds, ress, am_ref):
        am = am_ref[...]
        fs_src = _smap(lambda o, f: _mm(o, f), oh_src, fss)
        fd_dst = _smap(lambda o, f: _mm(o, f), oh_dst, fds)
        e = _smap(lambda a, b: jnp.where(a + b >= 0, a + b, (a + b) * 0.2),
                  fs_src, fd_dst)
        eb = _smap(lambda x: x.astype(BF16), e)
        logits_t = _smap(
            lambda x: lax.dot_general(am, x, (((0,), (1,)), ((), ())),
                                      preferred_element_type=F32), eb)
        gmax = _smap(lambda l: jnp.max(l, axis=(0, 1), keepdims=True),
                     logits_t)
        exl_t = _smap(lambda l, m: jnp.exp(l - m).astype(BF16),
                      logits_t, gmax)
        denom_t = _smap(lambda x, o: _mm(x, o), exl_t, oh_dst)
        denom_dst_t = _smap(
            lambda d, o: lax.dot_general(d.astype(BF16), o,
                                         (((1,), (1,)), ((), ())),
                                         preferred_element_type=F32),
            denom_t, oh_dst)
        a_t = _smap(lambda x, d: (x / jnp.maximum(d, 1e-9)).astype(BF16),
                    exl_t, denom_dst_t)
        em = em_ref[...]
        a_exp = _smap(lambda a: _mm_t(a, em), a_t)
        wgt = _smap(lambda a, f: (a * f).astype(BF16), a_exp, fs_src)
        rst = _smap(lambda o, w: _mm_t(o, w), oh_dst, wgt)
        return _smap(lambda r, q: jnp.maximum(r + q, 0.0), rst, ress)

    def pool(hs):
        cmax = _smap(lambda h: jnp.max(h, axis=0, keepdims=True), hs)
        ex = _smap(lambda h, c: jnp.exp(h - c), hs, cmax)
        newh = _smap(
            lambda x: (x * (1.0 / jnp.sum(x, axis=0, keepdims=True))
                       ).astype(BF16), ex)
        wg = wg_ref[...]
        g_t = _smap(
            lambda nh: lax.dot_general(wg, nh, (((0,), (1,)), ((), ())),
                                       preferred_element_type=F32) + b_gate,
            newh)
        gmx = _smap(lambda g: jnp.max(g, axis=1, keepdims=True), g_t)
        gex = _smap(lambda g, m: jnp.exp(g - m), g_t, gmx)
        gate_t = _smap(
            lambda x: (x * (1.0 / jnp.sum(x, axis=1, keepdims=True))
                       ).astype(BF16), gex)
        return _smap(lambda g, nh: _mm(g, nh), gate_t, newh)

    ws0 = ws0_ref[...]
    wd0 = wd0_ref[...]
    wr0 = wr0_ref[...]
    fs0 = _smap(lambda x: _mm(x, ws0).astype(BF16), xs)
    fd0 = _smap(lambda x: _mm(x, wd0).astype(BF16), xs)
    res0 = _smap(lambda x: _mm(x, wr0), xs)
    h1 = gat_layer(fs0, fd0, res0, am0_ref)
    hg = pool(h1)
    h1b = _smap(lambda h: h.astype(BF16), h1)
    ws1 = ws1_ref[...]
    wd1 = wd1_ref[...]
    fs1 = _smap(lambda h: (_mm(h, ws1) + b_src1).astype(BF16), h1b)
    fd1 = _smap(lambda h: (_mm(h, wd1) + b_dst1).astype(BF16), h1b)
    h2 = gat_layer(fs1, fd1, h1, am1_ref)
    hg2 = pool(h2)
    wl = wl_ref[...]
    for k in ks:
        out_ref[k] = _mm((hg[k] + hg2[k]).astype(BF16), wl) + b_lin


def _global_body(ge_ref, gf_ref, tf_ref, gnf_ref, wgcn_ref, wl2a_ref,
                 wl2b_ref, wclsa_ref, wclsb_ref, bp_ref, out_ref):
    ge = ge_ref[...]
    gf = gf_ref[...]
    gn = gf.shape[0]
    n_edges = ge.shape[0]
    chunk = 512
    n_chunks = n_edges // chunk
    bp = bp_ref[...]
    b_gcn = bp[0:1, 0:wgcn_ref.shape[1]]
    b_l2 = bp[1:2, 0:wl2a_ref.shape[1]]
    b_cls = bp[2:3, 0:out_ref.shape[1]]

    def onehots(c):
        sc = ge[c * chunk:(c + 1) * chunk, 0:1]
        dc = ge[c * chunk:(c + 1) * chunk, 1:2]
        n_iota = lax.broadcasted_iota(jnp.int32, (chunk, gn), 1)
        return (sc == n_iota).astype(F32), (dc == n_iota).astype(F32)

    ones_col = jnp.ones((chunk, 1), F32)
    deg_o = jnp.zeros((gn, 1), F32)
    deg_i = jnp.zeros((gn, 1), F32)
    for c in range(n_chunks):
        ohs, ohd = onehots(c)
        deg_o = deg_o + _mm_t(ohs, ones_col)
        deg_i = deg_i + _mm_t(ohd, ones_col)
    rsq_o = lax.rsqrt(jnp.maximum(deg_o, 1.0))
    rsq_i = lax.rsqrt(jnp.maximum(deg_i, 1.0))
    hsrc = gf * rsq_o
    agg = jnp.zeros_like(gf)
    for c in range(n_chunks):
        ohs, ohd = onehots(c)
        agg = agg + _mm_t(ohd, _mm(ohs, hsrc))
    agg = agg * rsq_i
    gcn = jnp.maximum(_mm(agg, wgcn_ref[...]) + b_gcn, 0.0)
    tra = _mm(gnf_ref[...], wl2a_ref[...]) + _mm(tf_ref[...], wl2b_ref[...]) + b_l2
    out_ref[...] = _mm(tra, wclsa_ref[...]) + _mm(gcn, wclsb_ref[...]) + b_cls


def _attn_mask(attn):
    n_heads, head_dim = attn.shape
    hd = n_heads * head_dim
    idx = jnp.arange(hd)
    return jnp.zeros((hd, n_heads), F32).at[idx, idx // head_dim].set(
        attn.reshape(-1))


def _full_spec(shape):
    nd = len(shape)
    return pl.BlockSpec(shape, lambda *_, _nd=nd: (0,) * _nd)


def kernel(sub_x, sub_edge_index, g_edge_index, g_feat, traFeat, params):
    p = params
    s, n, f_in = sub_x.shape
    e = sub_edge_index.shape[2]
    hd = p['W_src0'].shape[1]
    out_dim = p['W_lin'].shape[1]

    n_heads = p['attn0'].shape[0]
    xp = jnp.concatenate([sub_x, jnp.ones((s, n, 1), F32)],
                         axis=-1).astype(BF16)
    et = jnp.transpose(sub_edge_index.astype(jnp.int32), (0, 2, 1))
    ws0 = jnp.concatenate([p['W_src0'], p['b_src0'][None, :]],
                          axis=0).astype(BF16)
    wd0 = jnp.concatenate([p['W_dst0'], p['b_dst0'][None, :]],
                          axis=0).astype(BF16)
    wr0 = jnp.concatenate([p['res_W0'], p['res_b0'][None, :]],
                          axis=0).astype(BF16)
    am0 = _attn_mask(p['attn0']).astype(BF16)
    am1 = _attn_mask(p['attn1']).astype(BF16)
    em = (jnp.arange(hd)[None, :] // (hd // n_heads)
          == jnp.arange(n_heads)[:, None]).astype(BF16)
    bp = jnp.zeros((8, hd), F32)
    bp = bp.at[0, :].set(p['b_src1'])
    bp = bp.at[1, :].set(p['b_dst1'])
    bp = bp.at[2, 0].set(p['b_gate'][0])
    bp = bp.at[3, 0:out_dim].set(p['b_lin'])

    blk = 8
    gnf = pl.pallas_call(
        _sub_body,
        grid=(s // blk,),
        in_specs=[
            pl.BlockSpec((blk, n, f_in + 1), lambda i: (i, 0, 0)),
            pl.BlockSpec((blk, e, 2), lambda i: (i, 0, 0)),
            _full_spec(ws0.shape), _full_spec(wd0.shape),
            _full_spec(wr0.shape), _full_spec(am0.shape),
            _full_spec(p['W_src1'].shape), _full_spec(p['W_dst1'].shape),
            _full_spec(am1.shape), _full_spec(p['W_gate'].shape),
            _full_spec(p['W_lin'].shape), _full_spec(em.shape),
            _full_spec(bp.shape),
        ],
        out_specs=pl.BlockSpec((blk, 1, out_dim), lambda i: (i, 0, 0)),
        out_shape=jax.ShapeDtypeStruct((s, 1, out_dim), F32),
    )(xp, et, ws0, wd0, wr0, am0, p['W_src1'].astype(BF16),
      p['W_dst1'].astype(BF16), am1, p['W_gate'].astype(BF16),
      p['W_lin'].astype(BF16), em, bp)
    gnf = gnf.reshape(s, out_dim)

    gn, td = g_feat.shape
    geT = jnp.transpose(g_edge_index.astype(jnp.int32), (1, 0))
    wl2a = p['W_l2'][:out_dim, :]
    wl2b = p['W_l2'][out_dim:, :]
    h_dim = wl2a.shape[1]
    wclsa = p['W_cls'][:h_dim, :]
    wclsb = p['W_cls'][h_dim:, :]
    bp2 = jnp.zeros((4, max(td, h_dim)), F32)
    bp2 = bp2.at[0, 0:td].set(p['b_gcn'])
    bp2 = bp2.at[1, 0:h_dim].set(p['b_l2'])
    bp2 = bp2.at[2, 0:2].set(p['b_cls'])

    out = pl.pallas_call(
        _global_body,
        in_specs=[_full_spec(geT.shape), _full_spec(g_feat.shape),
                  _full_spec(traFeat.shape), _full_spec(gnf.shape),
                  _full_spec(p['W_gcn'].shape), _full_spec(wl2a.shape),
                  _full_spec(wl2b.shape), _full_spec(wclsa.shape),
                  _full_spec(wclsb.shape), _full_spec(bp2.shape)],
        out_specs=_full_spec((gn, 2)),
        out_shape=jax.ShapeDtypeStruct((gn, 2), F32),
    )(geT, g_feat, traFeat, gnf, p['W_gcn'], wl2a, wl2b, wclsa, wclsb, bp2)
    return out


# speedup vs baseline: 2.7370x; 1.0641x over previous
import jax
import jax.numpy as jnp
from jax import lax
from jax.experimental import pallas as pl
from jax.experimental.pallas import tpu as pltpu

F32 = jnp.float32
BF16 = jnp.bfloat16


def _mm(a, b):
    return lax.dot_general(a, b, (((1,), (0,)), ((), ())),
                           preferred_element_type=F32)


def _mm_t(a, b):
    return lax.dot_general(a, b, (((0,), (0,)), ((), ())),
                           preferred_element_type=F32)


def _smap(f, *ls):
    return [f(*xs) for xs in zip(*ls)]


def _sub_body(xp_ref, et_ref, ws0_ref, wd0_ref, wr0_ref, am0_ref,
              ws1_ref, wd1_ref, am1_ref, wg_ref, wl_ref, em_ref, bp_ref,
              out_ref):
    bp = bp_ref[...]
    b_src1 = bp[0:1, :]
    b_dst1 = bp[1:2, :]
    b_gate = bp[2:3, 0:1]
    b_lin = bp[3:4, 0:out_ref.shape[2]]
    blk = xp_ref.shape[0]
    n_nodes = xp_ref.shape[1]
    n_edges = et_ref.shape[1]
    ks = list(range(blk))

    xs = [xp_ref[k] for k in ks]
    n_iota = lax.broadcasted_iota(jnp.int32, (n_edges, n_nodes), 1)
    oh_src = [(et_ref[k][:, 0:1] == n_iota).astype(BF16) for k in ks]
    oh_dst = [(et_ref[k][:, 1:2] == n_iota).astype(BF16) for k in ks]

    def gat_layer(fss, fds, ress, am_ref):
        am = am_ref[...]
        fs_src = _smap(lambda o, f: _mm(o, f), oh_src, fss)
        fd_dst = _smap(lambda o, f: _mm(o, f), oh_dst, fds)
        e = _smap(lambda a, b: jnp.where(a + b >= 0, a + b, (a + b) * 0.2),
                  fs_src, fd_dst)
        eb = _smap(lambda x: x.astype(BF16), e)
        logits_t = _smap(
            lambda x: lax.dot_general(am, x, (((0,), (1,)), ((), ())),
                                      preferred_element_type=F32), eb)
        gmax = _smap(lambda l: jnp.max(l, axis=(0, 1), keepdims=True),
                     logits_t)
        exl_t = _smap(lambda l, m: jnp.exp(l - m).astype(BF16),
                      logits_t, gmax)
        denom_t = _smap(lambda x, o: _mm(x, o), exl_t, oh_dst)
        denom_dst_t = _smap(
            lambda d, o: lax.dot_general(d.astype(BF16), o,
                                         (((1,), (1,)), ((), ())),
                                         preferred_element_type=F32),
            denom_t, oh_dst)
        a_t = _smap(lambda x, d: (x / jnp.maximum(d, 1e-9)).astype(BF16),
                    exl_t, denom_dst_t)
        em = em_ref[...]
        a_exp = _smap(lambda a: _mm_t(a, em), a_t)
        wgt = _smap(lambda a, f: (a * f).astype(BF16), a_exp, fs_src)
        rst = _smap(lambda o, w: _mm_t(o, w), oh_dst, wgt)
        return _smap(lambda r, q: jnp.maximum(r + q, 0.0), rst, ress)

    def pool(hs):
        cmax = _smap(lambda h: jnp.max(h, axis=0, keepdims=True), hs)
        ex = _smap(lambda h, c: jnp.exp(h - c), hs, cmax)
        newh = _smap(
            lambda x: (x * (1.0 / jnp.sum(x, axis=0, keepdims=True))
                       ).astype(BF16), ex)
        wg = wg_ref[...]
        g_t = _smap(
            lambda nh: lax.dot_general(wg, nh, (((0,), (1,)), ((), ())),
                                       preferred_element_type=F32) + b_gate,
            newh)
        gmx = _smap(lambda g: jnp.max(g, axis=1, keepdims=True), g_t)
        gex = _smap(lambda g, m: jnp.exp(g - m), g_t, gmx)
        gate_t = _smap(
            lambda x: (x * (1.0 / jnp.sum(x, axis=1, keepdims=True))
                       ).astype(BF16), gex)
        return _smap(lambda g, nh: _mm(g, nh), gate_t, newh)

    ws0 = ws0_ref[...]
    wd0 = wd0_ref[...]
    wr0 = wr0_ref[...]
    fs0 = _smap(lambda x: _mm(x, ws0).astype(BF16), xs)
    fd0 = _smap(lambda x: _mm(x, wd0).astype(BF16), xs)
    res0 = _smap(lambda x: _mm(x, wr0), xs)
    h1 = gat_layer(fs0, fd0, res0, am0_ref)
    hg = pool(h1)
    h1b = _smap(lambda h: h.astype(BF16), h1)
    ws1 = ws1_ref[...]
    wd1 = wd1_ref[...]
    fs1 = _smap(lambda h: (_mm(h, ws1) + b_src1).astype(BF16), h1b)
    fd1 = _smap(lambda h: (_mm(h, wd1) + b_dst1).astype(BF16), h1b)
    h2 = gat_layer(fs1, fd1, h1, am1_ref)
    hg2 = pool(h2)
    wl = wl_ref[...]
    for k in ks:
        out_ref[k] = _mm((hg[k] + hg2[k]).astype(BF16), wl) + b_lin


def _global_body(ge_ref, gf_ref, tf_ref, gnf_ref, wgcn_ref, wl2a_ref,
                 wl2b_ref, wclsa_ref, wclsb_ref, bp_ref, out_ref):
    ge = ge_ref[...]
    gf = gf_ref[...]
    gn = gf.shape[0]
    n_edges = ge.shape[0]
    chunk = 512
    n_chunks = n_edges // chunk
    bp = bp_ref[...]
    b_gcn = bp[0:1, 0:wgcn_ref.shape[1]]
    b_l2 = bp[1:2, 0:wl2a_ref.shape[1]]
    b_cls = bp[2:3, 0:out_ref.shape[1]]

    def onehots(c):
        sc = ge[c * chunk:(c + 1) * chunk, 0:1]
        dc = ge[c * chunk:(c + 1) * chunk, 1:2]
        n_iota = lax.broadcasted_iota(jnp.int32, (chunk, gn), 1)
        return (sc == n_iota).astype(F32), (dc == n_iota).astype(F32)

    ones_col = jnp.ones((chunk, 1), F32)
    deg_o = jnp.zeros((gn, 1), F32)
    deg_i = jnp.zeros((gn, 1), F32)
    for c in range(n_chunks):
        ohs, ohd = onehots(c)
        deg_o = deg_o + _mm_t(ohs, ones_col)
        deg_i = deg_i + _mm_t(ohd, ones_col)
    rsq_o = lax.rsqrt(jnp.maximum(deg_o, 1.0))
    rsq_i = lax.rsqrt(jnp.maximum(deg_i, 1.0))
    hsrc = gf * rsq_o
    agg = jnp.zeros_like(gf)
    for c in range(n_chunks):
        ohs, ohd = onehots(c)
        agg = agg + _mm_t(ohd, _mm(ohs, hsrc))
    agg = agg * rsq_i
    gcn = jnp.maximum(_mm(agg, wgcn_ref[...]) + b_gcn, 0.0)
    tra = _mm(gnf_ref[...], wl2a_ref[...]) + _mm(tf_ref[...], wl2b_ref[...]) + b_l2
    out_ref[...] = _mm(tra, wclsa_ref[...]) + _mm(gcn, wclsb_ref[...]) + b_cls


def _attn_mask(attn):
    n_heads, head_dim = attn.shape
    hd = n_heads * head_dim
    idx = jnp.arange(hd)
    return jnp.zeros((hd, n_heads), F32).at[idx, idx // head_dim].set(
        attn.reshape(-1))


def _full_spec(shape):
    nd = len(shape)
    return pl.BlockSpec(shape, lambda *_, _nd=nd: (0,) * _nd)


def kernel(sub_x, sub_edge_index, g_edge_index, g_feat, traFeat, params):
    p = params
    s, n, f_in = sub_x.shape
    e = sub_edge_index.shape[2]
    hd = p['W_src0'].shape[1]
    out_dim = p['W_lin'].shape[1]

    n_heads = p['attn0'].shape[0]
    xp = jnp.concatenate([sub_x, jnp.ones((s, n, 1), F32)],
                         axis=-1).astype(BF16)
    et = jnp.transpose(sub_edge_index.astype(jnp.int32), (0, 2, 1))
    ws0 = jnp.concatenate([p['W_src0'], p['b_src0'][None, :]],
                          axis=0).astype(BF16)
    wd0 = jnp.concatenate([p['W_dst0'], p['b_dst0'][None, :]],
                          axis=0).astype(BF16)
    wr0 = jnp.concatenate([p['res_W0'], p['res_b0'][None, :]],
                          axis=0).astype(BF16)
    am0 = _attn_mask(p['attn0']).astype(BF16)
    am1 = _attn_mask(p['attn1']).astype(BF16)
    em = (jnp.arange(hd)[None, :] // (hd // n_heads)
          == jnp.arange(n_heads)[:, None]).astype(BF16)
    bp = jnp.zeros((8, hd), F32)
    bp = bp.at[0, :].set(p['b_src1'])
    bp = bp.at[1, :].set(p['b_dst1'])
    bp = bp.at[2, 0].set(p['b_gate'][0])
    bp = bp.at[3, 0:out_dim].set(p['b_lin'])

    blk = 16
    gnf = pl.pallas_call(
        _sub_body,
        grid=(s // blk,),
        in_specs=[
            pl.BlockSpec((blk, n, f_in + 1), lambda i: (i, 0, 0)),
            pl.BlockSpec((blk, e, 2), lambda i: (i, 0, 0)),
            _full_spec(ws0.shape), _full_spec(wd0.shape),
            _full_spec(wr0.shape), _full_spec(am0.shape),
            _full_spec(p['W_src1'].shape), _full_spec(p['W_dst1'].shape),
            _full_spec(am1.shape), _full_spec(p['W_gate'].shape),
            _full_spec(p['W_lin'].shape), _full_spec(em.shape),
            _full_spec(bp.shape),
        ],
        out_specs=pl.BlockSpec((blk, 1, out_dim), lambda i: (i, 0, 0)),
        out_shape=jax.ShapeDtypeStruct((s, 1, out_dim), F32),
    )(xp, et, ws0, wd0, wr0, am0, p['W_src1'].astype(BF16),
      p['W_dst1'].astype(BF16), am1, p['W_gate'].astype(BF16),
      p['W_lin'].astype(BF16), em, bp)
    gnf = gnf.reshape(s, out_dim)

    gn, td = g_feat.shape
    geT = jnp.transpose(g_edge_index.astype(jnp.int32), (1, 0))
    wl2a = p['W_l2'][:out_dim, :]
    wl2b = p['W_l2'][out_dim:, :]
    h_dim = wl2a.shape[1]
    wclsa = p['W_cls'][:h_dim, :]
    wclsb = p['W_cls'][h_dim:, :]
    bp2 = jnp.zeros((4, max(td, h_dim)), F32)
    bp2 = bp2.at[0, 0:td].set(p['b_gcn'])
    bp2 = bp2.at[1, 0:h_dim].set(p['b_l2'])
    bp2 = bp2.at[2, 0:2].set(p['b_cls'])

    out = pl.pallas_call(
        _global_body,
        in_specs=[_full_spec(geT.shape), _full_spec(g_feat.shape),
                  _full_spec(traFeat.shape), _full_spec(gnf.shape),
                  _full_spec(p['W_gcn'].shape), _full_spec(wl2a.shape),
                  _full_spec(wl2b.shape), _full_spec(wclsa.shape),
                  _full_spec(wclsb.shape), _full_spec(bp2.shape)],
        out_specs=_full_spec((gn, 2)),
        out_shape=jax.ShapeDtypeStruct((gn, 2), F32),
    )(geT, g_feat, traFeat, gnf, p['W_gcn'], wl2a, wl2b, wclsa, wclsb, bp2)
    return out
